# baseline (speedup 1.0000x reference)
import jax
import jax.numpy as jnp
from jax import lax
from jax.experimental import pallas as pl
from jax.experimental.pallas import tpu as pltpu
from jax.experimental.pallas import tpu_sc as plsc

_B, _S, _D = 4, 4096, 2048
_NC, _NS = 2, 16
_NW = _NC * _NS
_SPW = _S // _NW
_CH = 4
_NCHUNK = _SPW // _CH
_LANES = 16


def _sc_add(x_hbm, pe_hbm, out_hbm, pe_v, x_v, isem0, isem1, osem0, osem1):
    wid = lax.axis_index("s") * _NC + lax.axis_index("c")
    s0 = wid * _SPW
    isems = (isem0, isem1)
    osems = (osem0, osem1)

    def start_chunk(ci, buf):
        s = s0 + ci * _CH
        pltpu.async_copy(pe_hbm.at[pl.ds(s, _CH)], pe_v.at[buf], isems[buf])
        for b in range(_B):
            pltpu.async_copy(
                x_hbm.at[pl.ds(b * _S + s, _CH)], x_v.at[buf, b], isems[buf]
            )

    def wait_chunk(buf):
        pltpu.make_async_copy(
            pe_hbm.at[pl.ds(0, _CH)], pe_v.at[buf], isems[buf]
        ).wait()
        for b in range(_B):
            pltpu.make_async_copy(
                x_hbm.at[pl.ds(0, _CH)], x_v.at[buf, b], isems[buf]
            ).wait()

    def compute_store(ci, buf):
        def lane_body(j, _):
            sl = pl.ds(j * _LANES, _LANES)
            for i in range(_CH):
                pe_reg = pe_v[buf, i, sl]
                for b in range(_B):
                    x_v[buf, b, i, sl] = x_v[buf, b, i, sl] + pe_reg
            return _

        lax.fori_loop(0, _D // _LANES, lane_body, 0)
        s = s0 + ci * _CH
        for b in range(_B):
            pltpu.async_copy(
                x_v.at[buf, b], out_hbm.at[pl.ds(b * _S + s, _CH)], osems[buf]
            )

    def drain_store(buf):
        for b in range(_B):
            pltpu.make_async_copy(
                x_v.at[buf, b], out_hbm.at[pl.ds(0, _CH)], osems[buf]
            ).wait()

    start_chunk(0, 0)

    def pair_body(p, carry):
        ci = 2 * p

        @pl.when(p > 0)
        def _drain1():
            drain_store(1)
        start_chunk(ci + 1, 1)
        wait_chunk(0)
        compute_store(ci, 0)
        drain_store(0)

        @pl.when(ci + 2 < _NCHUNK)
        def _start_next():
            start_chunk(ci + 2, 0)

        wait_chunk(1)
        compute_store(ci + 1, 1)
        return carry

    lax.fori_loop(0, _NCHUNK // 2, pair_body, 0)
    drain_store(1)


def kernel(x, pe):
    B, S, D = x.shape
    xf = x.reshape(B * S, D)
    mesh = plsc.VectorSubcoreMesh(core_axis_name="c", subcore_axis_name="s")
    out = pl.kernel(
        _sc_add,
        mesh=mesh,
        out_type=jax.ShapeDtypeStruct((B * S, D), jnp.float32),
        scratch_types=[
            pltpu.VMEM((2, _CH, _D), jnp.float32),
            pltpu.VMEM((2, _B, _CH, _D), jnp.float32),
            pltpu.SemaphoreType.DMA,
            pltpu.SemaphoreType.DMA,
            pltpu.SemaphoreType.DMA,
            pltpu.SemaphoreType.DMA,
        ],
    )(xf, pe)
    return out.reshape(B, S, D)


# speedup vs baseline: 1.0200x; 1.0200x over previous
import jax
import jax.numpy as jnp
from jax import lax
from jax.experimental import pallas as pl
from jax.experimental.pallas import tpu as pltpu
from jax.experimental.pallas import tpu_sc as plsc

_B, _S, _D = 4, 4096, 2048
_NC, _NS = 2, 16
_NW = _NC * _NS
_SPW = _S // _NW
_CH = 2
_NCHUNK = _SPW // _CH
_NBUF = 4
_LANES = 16


def _sc_add(x_hbm, pe_hbm, out_hbm, pe_v, x_v,
            isem0, isem1, isem2, isem3, osem0, osem1, osem2, osem3):
    wid = lax.axis_index("s") * _NC + lax.axis_index("c")
    s0 = wid * _SPW
    isems = (isem0, isem1, isem2, isem3)
    osems = (osem0, osem1, osem2, osem3)

    def start_in(ci, buf):
        s = s0 + ci * _CH
        pltpu.async_copy(pe_hbm.at[pl.ds(s, _CH)], pe_v.at[buf], isems[buf])
        for b in range(_B):
            pltpu.async_copy(
                x_hbm.at[pl.ds(b * _S + s, _CH)], x_v.at[buf, b], isems[buf]
            )

    def wait_in(buf):
        pltpu.make_async_copy(
            pe_hbm.at[pl.ds(0, _CH)], pe_v.at[buf], isems[buf]
        ).wait()
        for b in range(_B):
            pltpu.make_async_copy(
                x_hbm.at[pl.ds(0, _CH)], x_v.at[buf, b], isems[buf]
            ).wait()

    def compute(buf):
        def lane_body(j, carry):
            sl = pl.ds(j * _LANES, _LANES)
            for i in range(_CH):
                pe_reg = pe_v[buf, i, sl]
                for b in range(_B):
                    x_v[buf, b, i, sl] = x_v[buf, b, i, sl] + pe_reg
            return carry

        lax.fori_loop(0, _D // _LANES, lane_body, 0)

    def start_out(ci, buf):
        s = s0 + ci * _CH
        for b in range(_B):
            pltpu.async_copy(
                x_v.at[buf, b], out_hbm.at[pl.ds(b * _S + s, _CH)], osems[buf]
            )

    def drain_out(buf):
        for b in range(_B):
            pltpu.make_async_copy(
                x_v.at[buf, b], out_hbm.at[pl.ds(0, _CH)], osems[buf]
            ).wait()

    start_in(0, 0)
    start_in(1, 1)

    def group_body(g, carry):
        for j in range(_NBUF):
            ci = _NBUF * g + j
            jj = (j + 2) % _NBUF

            @pl.when(ci - 2 >= 0)
            def _drain():
                drain_out(jj)

            @pl.when(ci + 2 < _NCHUNK)
            def _prefetch():
                start_in(ci + 2, jj)

            wait_in(j)
            compute(j)
            start_out(ci, j)
        return carry

    lax.fori_loop(0, _NCHUNK // _NBUF, group_body, 0)
    drain_out((_NCHUNK - 2) % _NBUF)
    drain_out((_NCHUNK - 1) % _NBUF)


def kernel(x, pe):
    B, S, D = x.shape
    xf = x.reshape(B * S, D)
    mesh = plsc.VectorSubcoreMesh(core_axis_name="c", subcore_axis_name="s")
    out = pl.kernel(
        _sc_add,
        mesh=mesh,
        out_type=jax.ShapeDtypeStruct((B * S, D), jnp.float32),
        scratch_types=[
            pltpu.VMEM((_NBUF, _CH, _D), jnp.float32),
            pltpu.VMEM((_NBUF, _B, _CH, _D), jnp.float32),
            pltpu.SemaphoreType.DMA,
            pltpu.SemaphoreType.DMA,
            pltpu.SemaphoreType.DMA,
            pltpu.SemaphoreType.DMA,
            pltpu.SemaphoreType.DMA,
            pltpu.SemaphoreType.DMA,
            pltpu.SemaphoreType.DMA,
            pltpu.SemaphoreType.DMA,
        ],
    )(xf, pe)
    return out.reshape(B, S, D)


# speedup vs baseline: 1.0246x; 1.0045x over previous
import jax
import jax.numpy as jnp
from jax import lax
from jax.experimental import pallas as pl
from jax.experimental.pallas import tpu as pltpu
from jax.experimental.pallas import tpu_sc as plsc

_B, _S, _D = 4, 4096, 2048
_NC, _NS = 2, 16
_NW = _NC * _NS
_SPW = _S // _NW
_CH = 2
_NCHUNK = _SPW // _CH
_NBUF = 4
_LANES = 16


def _sc_add(x_hbm, pe_hbm, out_hbm, pe_v, x_v,
            isem0, isem1, isem2, isem3, osem0, osem1, osem2, osem3):
    wid = lax.axis_index("s") * _NC + lax.axis_index("c")
    s0 = wid * _SPW
    isems = (isem0, isem1, isem2, isem3)
    osems = (osem0, osem1, osem2, osem3)

    def start_in(ci, buf):
        s = s0 + ci * _CH
        pltpu.async_copy(pe_hbm.at[pl.ds(s, _CH)], pe_v.at[buf], isems[buf])
        pltpu.async_copy(
            x_hbm.at[:, pl.ds(s, _CH)], x_v.at[buf], isems[buf]
        )

    def wait_in(buf):
        pltpu.make_async_copy(
            pe_hbm.at[pl.ds(0, _CH)], pe_v.at[buf], isems[buf]
        ).wait()
        pltpu.make_async_copy(
            x_hbm.at[:, pl.ds(0, _CH)], x_v.at[buf], isems[buf]
        ).wait()

    def compute(buf):
        def lane_body(j, carry):
            sl = pl.ds(j * _LANES, _LANES)
            for i in range(_CH):
                pe_reg = pe_v[buf, i, sl]
                for b in range(_B):
                    x_v[buf, b, i, sl] = x_v[buf, b, i, sl] + pe_reg
            return carry

        lax.fori_loop(0, _D // _LANES, lane_body, 0)

    def start_out(ci, buf):
        s = s0 + ci * _CH
        pltpu.async_copy(
            x_v.at[buf], out_hbm.at[:, pl.ds(s, _CH)], osems[buf]
        )

    def drain_out(buf):
        pltpu.make_async_copy(
            x_v.at[buf], out_hbm.at[:, pl.ds(0, _CH)], osems[buf]
        ).wait()

    start_in(0, 0)
    start_in(1, 1)

    def group_body(g, carry):
        for j in range(_NBUF):
            ci = _NBUF * g + j
            jj = (j + 2) % _NBUF

            @pl.when(ci - 2 >= 0)
            def _drain():
                drain_out(jj)

            @pl.when(ci + 2 < _NCHUNK)
            def _prefetch():
                start_in(ci + 2, jj)

            wait_in(j)
            compute(j)
            start_out(ci, j)
        return carry

    lax.fori_loop(0, _NCHUNK // _NBUF, group_body, 0)
    drain_out((_NCHUNK - 2) % _NBUF)
    drain_out((_NCHUNK - 1) % _NBUF)


def kernel(x, pe):
    B, S, D = x.shape
    mesh = plsc.VectorSubcoreMesh(core_axis_name="c", subcore_axis_name="s")
    return pl.kernel(
        _sc_add,
        mesh=mesh,
        out_type=jax.ShapeDtypeStruct((B, S, D), jnp.float32),
        scratch_types=[
            pltpu.VMEM((_NBUF, _CH, _D), jnp.float32),
            pltpu.VMEM((_NBUF, _B, _CH, _D), jnp.float32),
            pltpu.SemaphoreType.DMA,
            pltpu.SemaphoreType.DMA,
            pltpu.SemaphoreType.DMA,
            pltpu.SemaphoreType.DMA,
            pltpu.SemaphoreType.DMA,
            pltpu.SemaphoreType.DMA,
            pltpu.SemaphoreType.DMA,
            pltpu.SemaphoreType.DMA,
        ],
    )(x, pe)
